# baseline (speedup 1.0000x reference)
import jax
import jax.numpy as jnp
from jax import lax
from jax.experimental import pallas as pl
from jax.experimental.pallas import tpu as pltpu
from jax.experimental.pallas import tpu_sc as plsc

NC, NS = 2, 16
NW = NC * NS
KR = 6
KS = 3
LG = 6
H, D, LANES = 200, 64, 128
TW = 64
SLP = 129
TRS, RS = 8, 8


def _emb_body(idx_hbm, tbl_hbm, out_hbm, idx_v, rows_v, slab_v, gsem, osem):
    w = lax.axis_index("s") * NC + lax.axis_index("c")
    iota = lax.iota(jnp.int32, 16)

    pltpu.sync_copy(idx_hbm.at[:, w], idx_v)

    def start_gather(h, b):
        tr8, r8 = h // RS, h % RS
        pltpu.async_copy(tbl_hbm.at[idx_v.at[tr8, r8]], rows_v.at[b],
                         gsem.at[b])

    def wait_gather(b):
        pltpu.make_async_copy(tbl_hbm.at[idx_v.at[0, 0]], rows_v.at[b],
                              gsem.at[b]).wait()

    def transpose_step(rb, sb):
        @plsc.parallel_loop(0, LANES, unroll=4)
        def _(b):
            bv = jnp.full((16,), 0, jnp.int32) + b
            for f0 in range(0, D, 16):
                v = rows_v[rb, b, pl.ds(f0, 16)]
                plsc.store_scatter(slab_v.at[sb],
                                   [(f0 + iota) // RS, (f0 + iota) % RS, bv],
                                   v)

    def start_out(h, sb):
        def tr_body(tr, carry):
            pltpu.async_copy(slab_v.at[sb, tr, :, pl.ds(0, LANES)],
                             out_hbm.at[h, tr, w], osem.at[sb])
            return carry

        lax.fori_loop(0, TRS, tr_body, 0)

    def wait_out(sb):
        def tr_body(tr, carry):
            pltpu.make_async_copy(slab_v.at[sb, tr, :, pl.ds(0, LANES)],
                                  out_hbm.at[0, tr, w], osem.at[sb]).wait()
            return carry

        lax.fori_loop(0, TRS, tr_body, 0)

    for b in range(LG):
        start_gather(b, b)

    def step(h, carry):
        rb = h % KR
        sb = h % KS

        @pl.when(h >= KS)
        def _():
            wait_out(sb)

        wait_gather(rb)
        transpose_step(rb, sb)

        @pl.when(h + LG < H)
        def _():
            start_gather(h + LG, rb)

        start_out(h, sb)
        return carry

    lax.fori_loop(0, H, step, 0)
    for k in range(KS):
        wait_out((H - KS + k) % KS)


def kernel(indices, table):
    B, _ = indices.shape
    idx4 = ((indices.astype(jnp.int32) * 2).T
            .reshape(H // RS, RS, NW, LANES).transpose(0, 2, 1, 3))
    tbl2 = jnp.concatenate([table, jnp.zeros_like(table)],
                           axis=1).reshape(2 * table.shape[0], D)

    run = pl.kernel(
        _emb_body,
        out_type=jax.ShapeDtypeStruct((H, TRS, NW, RS, LANES), jnp.float32),
        mesh=plsc.VectorSubcoreMesh(core_axis_name="c", subcore_axis_name="s"),
        compiler_params=pltpu.CompilerParams(use_tc_tiling_on_sc=False,
                                             needs_layout_passes=False),
        scratch_types=[
            pltpu.VMEM((H // RS, RS, LANES), jnp.int32),
            pltpu.VMEM((KR, LANES, TW), jnp.float32),
            pltpu.VMEM((KS, TRS, RS, SLP), jnp.float32),
            pltpu.SemaphoreType.DMA((KR,)),
            pltpu.SemaphoreType.DMA((KS,)),
        ],
    )
    out6 = run(idx4, tbl2)
    return out6.transpose(2, 4, 0, 1, 3).reshape(B, H, D)
